# baseline (speedup 1.0000x reference)
import functools

import jax
import jax.numpy as jnp
from jax import lax
from jax.experimental import pallas as pl
from jax.experimental.pallas import tpu as pltpu
from jax.experimental.pallas import tpu_sc as plsc

_N = 4096
_D = 128
_G = 32
_K = 16
_OUT = 64
_NB = 5
_TR = 128
_TCW = 128
_NT = _N // _TCW
_BIG = 3.0e38

_TOTROWS = _NB * _N
_SC_CHUNK = 128
_SC_WORKERS = 32
_RPW = _TOTROWS // _SC_WORKERS


def _sh_body(x_ref, w_ref, b_ref, o_ref):
    o_ref[0] = (
        jnp.dot(x_ref[0], w_ref[0], preferred_element_type=jnp.float32)
        + b_ref[0]
    )


def _sh_call(x5, wsh, bsh):
    return pl.pallas_call(
        _sh_body,
        grid=(_NB,),
        in_specs=[
            pl.BlockSpec((1, _N, _D), lambda b: (b, 0, 0)),
            pl.BlockSpec((1, _D, 4), lambda b: (b, 0, 0)),
            pl.BlockSpec((1, 1, 4), lambda b: (b, 0, 0)),
        ],
        out_specs=pl.BlockSpec((1, _N, 4), lambda b: (b, 0, 0)),
        out_shape=jax.ShapeDtypeStruct((_NB, _N, 4), jnp.float32),
    )(x5, wsh, bsh)


def _gravnet_body(shr_ref, sht_ref, br_ref, bc_ref, x_ref, wo1_ref, wo2_ref,
                  bo2_ref, y_ref, d2_ref):
    shr = shr_ref[0]
    bi = br_ref[0]
    six = shr[:, 0:1]
    siy = shr[:, 1:2]
    siz = shr[:, 2:3]

    bcall = bc_ref[0]
    lo_g = jnp.min(bi)
    hi_g = jnp.max(bi)
    col_lo = jnp.sum((bcall < lo_g).astype(jnp.int32))
    col_hi = jnp.sum((bcall <= hi_g).astype(jnp.int32))
    t0 = col_lo // _TCW
    t1 = (col_hi + _TCW - 1) // _TCW

    def fill_body(c, carry):
        sjx = sht_ref[0, 0:1, c, :]
        sjy = sht_ref[0, 1:2, c, :]
        sjz = sht_ref[0, 2:3, c, :]
        bj = bc_ref[0, c]
        dx = six - sjx
        dy = siy - sjy
        dz = siz - sjz
        d2 = (dx * dx + dy * dy) + dz * dz
        d2 = jnp.where(bi != bj, jnp.float32(1e10), d2)
        d2_ref[c] = d2
        return carry

    lax.fori_loop(t0, t1, fill_body, 0)

    prev = jnp.full((_TR, 1), -_BIG, jnp.float32)
    for _ in range(_K):
        def kth_body(c, m, lim=prev):
            tile = d2_ref[c]
            vals = jnp.where(tile > lim, tile, _BIG)
            return jnp.minimum(m, jnp.min(vals, axis=1, keepdims=True))
        prev = lax.fori_loop(t0, t1, kth_body,
                             jnp.full((_TR, 1), _BIG, jnp.float32))
    thr = prev

    def agg_body(c, carry):
        sacc, macc = carry
        tile = d2_ref[c]
        hj = sht_ref[0, 3:4, c, :]
        msg = hj * jnp.exp(-10.0 * tile)
        sel = tile <= thr
        sacc = sacc + jnp.sum(jnp.where(sel, msg, 0.0), axis=1, keepdims=True)
        macc = jnp.maximum(
            macc, jnp.max(jnp.where(sel, msg, -_BIG), axis=1, keepdims=True))
        return sacc, macc

    sacc, macc = lax.fori_loop(
        t0, t1, agg_body,
        (jnp.zeros((_TR, 1), jnp.float32),
         jnp.full((_TR, 1), -_BIG, jnp.float32)))

    mean = sacc * (1.0 / _K)
    wo2 = wo2_ref[0]
    v = (jnp.dot(x_ref[0], wo1_ref[0], preferred_element_type=jnp.float32)
         + mean * wo2[0:1, :] + macc * wo2[1:2, :] + bo2_ref[0])
    y_ref[0] = v


def _gravnet_call(shr, sht4, br, bc4, x5, wo1, wo2, bo2):
    return pl.pallas_call(
        _gravnet_body,
        grid=(_NB, _N // _TR),
        in_specs=[
            pl.BlockSpec((1, _TR, 4), lambda b, r: (b, r, 0)),
            pl.BlockSpec((1, 4, _NT, _TCW), lambda b, r: (b, 0, 0, 0)),
            pl.BlockSpec((1, _TR, 1), lambda b, r: (b, r, 0)),
            pl.BlockSpec((1, _NT, 1, _TCW), lambda b, r: (b, 0, 0, 0)),
            pl.BlockSpec((1, _TR, _D), lambda b, r: (b, r, 0)),
            pl.BlockSpec((1, _D, _OUT), lambda b, r: (b, 0, 0)),
            pl.BlockSpec((1, 2, _OUT), lambda b, r: (b, 0, 0)),
            pl.BlockSpec((1, 1, _OUT), lambda b, r: (b, 0, 0)),
        ],
        out_specs=pl.BlockSpec((1, _TR, _OUT), lambda b, r: (b, r, 0)),
        out_shape=jax.ShapeDtypeStruct((_NB, _N, _OUT), jnp.float32),
        scratch_shapes=[pltpu.VMEM((_NT, _TR, _TCW), jnp.float32)],
    )(shr, sht4, br, bc4, x5, wo1, wo2, bo2)


_SC_ROWS = _TOTROWS * _OUT // 128
_RPW_SC = _SC_ROWS // _SC_WORKERS


def _eluw_sc(v2, wb2):
    mesh = plsc.VectorSubcoreMesh(core_axis_name="c", subcore_axis_name="s")

    @functools.partial(
        pl.kernel,
        mesh=mesh,
        out_type=jax.ShapeDtypeStruct((_SC_ROWS, 128), jnp.float32),
        scratch_types=[
            pltpu.VMEM((_RPW_SC, 128), jnp.float32),
            pltpu.VMEM((_RPW_SC, 128), jnp.float32),
        ],
    )
    def run(v_hbm, w_hbm, t_hbm, v_v, w_v):
        cid = lax.axis_index("c")
        sid = lax.axis_index("s")
        wid = sid * 2 + cid
        base = wid * _RPW_SC
        pltpu.sync_copy(v_hbm.at[pl.ds(base, _RPW_SC)], v_v)
        pltpu.sync_copy(w_hbm.at[pl.ds(base, _RPW_SC)], w_v)

        def body(i, carry):
            for c in range(128 // 16):
                sl = pl.ds(c * 16, 16)
                x = v_v[i, sl]
                e = jnp.where(x > 0.0, x, jnp.exp(x) - 1.0)
                v_v[i, sl] = e * w_v[i, sl]
            return carry

        lax.fori_loop(0, _RPW_SC, body, 0)
        pltpu.sync_copy(v_v, t_hbm.at[pl.ds(base, _RPW_SC)])

    return run(v2, wb2)


_TRC = 2048


def _head_body(t_ref, oh_ref, b_ref, o_ref):
    r = pl.program_id(0)

    @pl.when(r == 0)
    def _():
        o_ref[...] = jnp.broadcast_to(b_ref[...], (_G, 1))

    z = jnp.dot(t_ref[...], jnp.ones((_OUT, 1), jnp.float32),
                preferred_element_type=jnp.float32)
    contrib = jax.lax.dot_general(
        oh_ref[...], z, (((0,), (0,)), ((), ())),
        preferred_element_type=jnp.float32)
    o_ref[...] = o_ref[...] + contrib


def _head_call(t2, oh, bh):
    return pl.pallas_call(
        _head_body,
        grid=(_TOTROWS // _TRC,),
        in_specs=[
            pl.BlockSpec((_TRC, _OUT), lambda r: (r, 0)),
            pl.BlockSpec((_TRC, _G), lambda r: (r, 0)),
            pl.BlockSpec((1, 1), lambda r: (0, 0)),
        ],
        out_specs=pl.BlockSpec((_G, 1), lambda r: (0, 0)),
        out_shape=jax.ShapeDtypeStruct((_G, 1), jnp.float32),
    )(t2, oh, bh)


def kernel(x_SiPix, batch_SiPix, x_Crystal, batch_Crystal, x_WSi, batch_WSi,
           x_PbSi, batch_PbSi, x_PbScint, batch_PbScint, params):
    xs = jnp.stack([x_SiPix, x_Crystal, x_WSi, x_PbSi, x_PbScint])
    bs = jnp.stack([batch_SiPix, batch_Crystal, batch_WSi, batch_PbSi,
                    batch_PbScint]).astype(jnp.int32)
    convs = [params['conv1'], params['conv1'], params['conv2'],
             params['conv3'], params['conv4']]
    wsh = jnp.stack([jnp.concatenate([c['Ws'], c['Wh']], axis=1)
                     for c in convs])
    bsh = jnp.stack([jnp.concatenate([c['bs'], c['bh']])[None, :]
                     for c in convs])
    wo1 = jnp.stack([c['Wo1'] for c in convs])
    wo2 = jnp.stack([c['Wo2'] for c in convs])
    bo2 = jnp.stack([c['bo2'][None, :] for c in convs])

    sh = _sh_call(xs, wsh, bsh)
    sht4 = jnp.swapaxes(sh, 1, 2).reshape(_NB, 4, _NT, _TCW)
    br = bs[:, :, None]
    bc4 = bs.reshape(_NB, _NT, 1, _TCW)

    v = _gravnet_call(sh, sht4, br, bc4, xs, wo1, wo2, bo2)

    v2 = v.reshape(_SC_ROWS, 128)
    wb = params['out']['W'].reshape(_NB, 1, _OUT)
    wb2 = jnp.broadcast_to(wb, (_NB, _N, _OUT)).reshape(_SC_ROWS, 128)
    t2 = _eluw_sc(v2, wb2)

    oh = (bs.reshape(_TOTROWS)[:, None]
          == jnp.arange(_G, dtype=jnp.int32)[None, :]).astype(jnp.float32)
    bh = params['out']['b'].reshape(1, 1)
    return _head_call(t2.reshape(_TOTROWS, _OUT), oh, bh)


# speedup vs baseline: 1.5699x; 1.5699x over previous
import functools

import jax
import jax.numpy as jnp
from jax import lax
from jax.experimental import pallas as pl
from jax.experimental.pallas import tpu as pltpu
from jax.experimental.pallas import tpu_sc as plsc

_N = 4096
_D = 128
_G = 32
_K = 16
_OUT = 64
_NB = 5
_TR = 256
_TCW = 256
_NT = _N // _TCW
_BIG = 3.0e38

_TOTROWS = _NB * _N
_SC_CHUNK = 128
_SC_WORKERS = 32
_RPW = _TOTROWS // _SC_WORKERS


def _sh_body(x_ref, w_ref, b_ref, o_ref):
    o_ref[0] = (
        jnp.dot(x_ref[0], w_ref[0], preferred_element_type=jnp.float32)
        + b_ref[0]
    )


def _sh_call(x5, wsh, bsh):
    return pl.pallas_call(
        _sh_body,
        grid=(_NB,),
        in_specs=[
            pl.BlockSpec((1, _N, _D), lambda b: (b, 0, 0)),
            pl.BlockSpec((1, _D, 4), lambda b: (b, 0, 0)),
            pl.BlockSpec((1, 1, 4), lambda b: (b, 0, 0)),
        ],
        out_specs=pl.BlockSpec((1, _N, 4), lambda b: (b, 0, 0)),
        out_shape=jax.ShapeDtypeStruct((_NB, _N, 4), jnp.float32),
    )(x5, wsh, bsh)


def _gravnet_body(shr_ref, sht_ref, br_ref, bc_ref, x_ref, wo1_ref, wo2_ref,
                  bo2_ref, y_ref, d2_ref):
    shr = shr_ref[0]
    bi = br_ref[0]
    six = shr[:, 0:1]
    siy = shr[:, 1:2]
    siz = shr[:, 2:3]

    bcall = bc_ref[0]
    lo_g = jnp.min(bi)
    hi_g = jnp.max(bi)
    col_lo = jnp.sum((bcall < lo_g).astype(jnp.int32))
    col_hi = jnp.sum((bcall <= hi_g).astype(jnp.int32))
    t0 = col_lo // _TCW
    t1 = (col_hi + _TCW - 1) // _TCW

    def tile_d2(c):
        sjx = sht_ref[0, 0:1, c, :]
        sjy = sht_ref[0, 1:2, c, :]
        sjz = sht_ref[0, 2:3, c, :]
        bj = bc_ref[0, c]
        dx = six - sjx
        dy = siy - sjy
        dz = siz - sjz
        d2 = (dx * dx + dy * dy) + dz * dz
        return jnp.where(bi != bj, jnp.float32(1e10), d2)

    def fast_path(_):
        t0b = jnp.minimum(t0, _NT - 2)
        d0 = tile_d2(t0b)
        d1 = tile_d2(t0b + 1)
        prev = jnp.full((_TR, 1), -_BIG, jnp.float32)
        for _ in range(_K):
            v0 = jnp.min(jnp.where(d0 > prev, d0, _BIG),
                         axis=1, keepdims=True)
            v1 = jnp.min(jnp.where(d1 > prev, d1, _BIG),
                         axis=1, keepdims=True)
            prev = jnp.minimum(v0, v1)
        thr = prev
        sacc = jnp.zeros((_TR, 1), jnp.float32)
        macc = jnp.full((_TR, 1), -_BIG, jnp.float32)
        for c, d in ((t0b, d0), (t0b + 1, d1)):
            hj = sht_ref[0, 3:4, c, :]
            msg = hj * jnp.exp(-10.0 * d)
            sel = d <= thr
            sacc = sacc + jnp.sum(jnp.where(sel, msg, 0.0),
                                  axis=1, keepdims=True)
            macc = jnp.maximum(
                macc,
                jnp.max(jnp.where(sel, msg, -_BIG), axis=1, keepdims=True))
        return sacc, macc

    def slow_path(_):
        def fill_body(c, carry):
            d2_ref[c] = tile_d2(c)
            return carry

        lax.fori_loop(t0, t1, fill_body, 0)

        prev = jnp.full((_TR, 1), -_BIG, jnp.float32)
        for _ in range(_K):
            def kth_body(c, m, lim=prev):
                tile = d2_ref[c]
                vals = jnp.where(tile > lim, tile, _BIG)
                return jnp.minimum(m, jnp.min(vals, axis=1, keepdims=True))
            prev = lax.fori_loop(t0, t1, kth_body,
                                 jnp.full((_TR, 1), _BIG, jnp.float32))
        thr = prev

        def agg_body(c, carry):
            sacc, macc = carry
            tile = d2_ref[c]
            hj = sht_ref[0, 3:4, c, :]
            msg = hj * jnp.exp(-10.0 * tile)
            sel = tile <= thr
            sacc = sacc + jnp.sum(jnp.where(sel, msg, 0.0),
                                  axis=1, keepdims=True)
            macc = jnp.maximum(
                macc,
                jnp.max(jnp.where(sel, msg, -_BIG), axis=1, keepdims=True))
            return sacc, macc

        return lax.fori_loop(
            t0, t1, agg_body,
            (jnp.zeros((_TR, 1), jnp.float32),
             jnp.full((_TR, 1), -_BIG, jnp.float32)))

    sacc, macc = lax.cond(t1 - jnp.minimum(t0, _NT - 2) <= 2,
                          fast_path, slow_path, 0)

    mean = sacc * (1.0 / _K)
    wo2 = wo2_ref[0]
    v = (jnp.dot(x_ref[0], wo1_ref[0], preferred_element_type=jnp.float32)
         + mean * wo2[0:1, :] + macc * wo2[1:2, :] + bo2_ref[0])
    y_ref[0] = v


def _gravnet_call(shr, sht4, br, bc4, x5, wo1, wo2, bo2):
    return pl.pallas_call(
        _gravnet_body,
        grid=(_NB, _N // _TR),
        in_specs=[
            pl.BlockSpec((1, _TR, 4), lambda b, r: (b, r, 0)),
            pl.BlockSpec((1, 4, _NT, _TCW), lambda b, r: (b, 0, 0, 0)),
            pl.BlockSpec((1, _TR, 1), lambda b, r: (b, r, 0)),
            pl.BlockSpec((1, _NT, 1, _TCW), lambda b, r: (b, 0, 0, 0)),
            pl.BlockSpec((1, _TR, _D), lambda b, r: (b, r, 0)),
            pl.BlockSpec((1, _D, _OUT), lambda b, r: (b, 0, 0)),
            pl.BlockSpec((1, 2, _OUT), lambda b, r: (b, 0, 0)),
            pl.BlockSpec((1, 1, _OUT), lambda b, r: (b, 0, 0)),
        ],
        out_specs=pl.BlockSpec((1, _TR, _OUT), lambda b, r: (b, r, 0)),
        out_shape=jax.ShapeDtypeStruct((_NB, _N, _OUT), jnp.float32),
        scratch_shapes=[pltpu.VMEM((_NT, _TR, _TCW), jnp.float32)],
    )(shr, sht4, br, bc4, x5, wo1, wo2, bo2)


_SC_ROWS = _TOTROWS * _OUT // 128
_RPW_SC = _SC_ROWS // _SC_WORKERS


def _eluw_sc(v2, wb2):
    mesh = plsc.VectorSubcoreMesh(core_axis_name="c", subcore_axis_name="s")

    @functools.partial(
        pl.kernel,
        mesh=mesh,
        out_type=jax.ShapeDtypeStruct((_SC_ROWS, 128), jnp.float32),
        scratch_types=[
            pltpu.VMEM((_RPW_SC, 128), jnp.float32),
            pltpu.VMEM((_RPW_SC, 128), jnp.float32),
        ],
    )
    def run(v_hbm, w_hbm, t_hbm, v_v, w_v):
        cid = lax.axis_index("c")
        sid = lax.axis_index("s")
        wid = sid * 2 + cid
        base = wid * _RPW_SC
        pltpu.sync_copy(v_hbm.at[pl.ds(base, _RPW_SC)], v_v)
        pltpu.sync_copy(w_hbm.at[pl.ds(base, _RPW_SC)], w_v)

        def body(i, carry):
            for c in range(128 // 16):
                sl = pl.ds(c * 16, 16)
                x = v_v[i, sl]
                e = jnp.where(x > 0.0, x, jnp.exp(x) - 1.0)
                v_v[i, sl] = e * w_v[i, sl]
            return carry

        lax.fori_loop(0, _RPW_SC, body, 0)
        pltpu.sync_copy(v_v, t_hbm.at[pl.ds(base, _RPW_SC)])

    return run(v2, wb2)


_TRC = 2048


def _head_body(t_ref, oh_ref, b_ref, o_ref):
    r = pl.program_id(0)

    @pl.when(r == 0)
    def _():
        o_ref[...] = jnp.broadcast_to(b_ref[...], (_G, 1))

    z = jnp.dot(t_ref[...], jnp.ones((_OUT, 1), jnp.float32),
                preferred_element_type=jnp.float32)
    contrib = jax.lax.dot_general(
        oh_ref[...], z, (((0,), (0,)), ((), ())),
        preferred_element_type=jnp.float32)
    o_ref[...] = o_ref[...] + contrib


def _head_call(t2, oh, bh):
    return pl.pallas_call(
        _head_body,
        grid=(_TOTROWS // _TRC,),
        in_specs=[
            pl.BlockSpec((_TRC, _OUT), lambda r: (r, 0)),
            pl.BlockSpec((_TRC, _G), lambda r: (r, 0)),
            pl.BlockSpec((1, 1), lambda r: (0, 0)),
        ],
        out_specs=pl.BlockSpec((_G, 1), lambda r: (0, 0)),
        out_shape=jax.ShapeDtypeStruct((_G, 1), jnp.float32),
    )(t2, oh, bh)


def kernel(x_SiPix, batch_SiPix, x_Crystal, batch_Crystal, x_WSi, batch_WSi,
           x_PbSi, batch_PbSi, x_PbScint, batch_PbScint, params):
    xs = jnp.stack([x_SiPix, x_Crystal, x_WSi, x_PbSi, x_PbScint])
    bs = jnp.stack([batch_SiPix, batch_Crystal, batch_WSi, batch_PbSi,
                    batch_PbScint]).astype(jnp.int32)
    convs = [params['conv1'], params['conv1'], params['conv2'],
             params['conv3'], params['conv4']]
    wsh = jnp.stack([jnp.concatenate([c['Ws'], c['Wh']], axis=1)
                     for c in convs])
    bsh = jnp.stack([jnp.concatenate([c['bs'], c['bh']])[None, :]
                     for c in convs])
    wo1 = jnp.stack([c['Wo1'] for c in convs])
    wo2 = jnp.stack([c['Wo2'] for c in convs])
    bo2 = jnp.stack([c['bo2'][None, :] for c in convs])

    sh = _sh_call(xs, wsh, bsh)
    sht4 = jnp.swapaxes(sh, 1, 2).reshape(_NB, 4, _NT, _TCW)
    br = bs[:, :, None]
    bc4 = bs.reshape(_NB, _NT, 1, _TCW)

    v = _gravnet_call(sh, sht4, br, bc4, xs, wo1, wo2, bo2)

    v2 = v.reshape(_SC_ROWS, 128)
    wb = params['out']['W'].reshape(_NB, 1, _OUT)
    wb2 = jnp.broadcast_to(wb, (_NB, _N, _OUT)).reshape(_SC_ROWS, 128)
    t2 = _eluw_sc(v2, wb2)

    oh = (bs.reshape(_TOTROWS)[:, None]
          == jnp.arange(_G, dtype=jnp.int32)[None, :]).astype(jnp.float32)
    bh = params['out']['b'].reshape(1, 1)
    return _head_call(t2.reshape(_TOTROWS, _OUT), oh, bh)


# speedup vs baseline: 1.6036x; 1.0215x over previous
import functools

import jax
import jax.numpy as jnp
from jax import lax
from jax.experimental import pallas as pl
from jax.experimental.pallas import tpu as pltpu
from jax.experimental.pallas import tpu_sc as plsc

_N = 4096
_D = 128
_G = 32
_K = 16
_OUT = 64
_NB = 5
_TR = 256
_TCW = 256
_NT = _N // _TCW
_BIG = 3.0e38

_TOTROWS = _NB * _N
_SC_CHUNK = 128
_SC_WORKERS = 32
_RPW = _TOTROWS // _SC_WORKERS


def _sh_body(x_ref, w_ref, b_ref, o_ref):
    o_ref[0] = (
        jnp.dot(x_ref[0], w_ref[0], preferred_element_type=jnp.float32)
        + b_ref[0]
    )


def _sh_call(x5, wsh, bsh):
    return pl.pallas_call(
        _sh_body,
        grid=(_NB,),
        in_specs=[
            pl.BlockSpec((1, _N, _D), lambda b: (b, 0, 0)),
            pl.BlockSpec((1, _D, 4), lambda b: (b, 0, 0)),
            pl.BlockSpec((1, 1, 4), lambda b: (b, 0, 0)),
        ],
        out_specs=pl.BlockSpec((1, _N, 4), lambda b: (b, 0, 0)),
        out_shape=jax.ShapeDtypeStruct((_NB, _N, 4), jnp.float32),
    )(x5, wsh, bsh)


def _gravnet_body(shr_ref, sht_ref, br_ref, bc_ref, x_ref, wo1_ref, wo2_ref,
                  bo2_ref, y_ref, d2_ref):
    shr = shr_ref[0]
    bi = br_ref[0]
    six = shr[:, 0:1]
    siy = shr[:, 1:2]
    siz = shr[:, 2:3]

    bcall = bc_ref[0]
    lo_g = jnp.min(bi)
    hi_g = jnp.max(bi)
    col_lo = jnp.sum((bcall < lo_g).astype(jnp.int32))
    col_hi = jnp.sum((bcall <= hi_g).astype(jnp.int32))
    t0 = col_lo // _TCW
    t1 = (col_hi + _TCW - 1) // _TCW

    def tile_d2(c):
        sjx = sht_ref[0, 0:1, c, :]
        sjy = sht_ref[0, 1:2, c, :]
        sjz = sht_ref[0, 2:3, c, :]
        bj = bc_ref[0, c]
        dx = six - sjx
        dy = siy - sjy
        dz = siz - sjz
        d2 = (dx * dx + dy * dy) + dz * dz
        return jnp.where(bi != bj, jnp.float32(1e10), d2)

    def fast_path(_):
        t0b = jnp.minimum(t0, _NT - 2)
        d0 = tile_d2(t0b)
        d1 = tile_d2(t0b + 1)
        prev = jnp.full((_TR, 1), -_BIG, jnp.float32)
        for _ in range(_K):
            v0 = jnp.min(jnp.where(d0 > prev, d0, _BIG),
                         axis=1, keepdims=True)
            v1 = jnp.min(jnp.where(d1 > prev, d1, _BIG),
                         axis=1, keepdims=True)
            prev = jnp.minimum(v0, v1)
        thr = prev
        sacc = jnp.zeros((_TR, 1), jnp.float32)
        macc = jnp.full((_TR, 1), -_BIG, jnp.float32)
        for c, d in ((t0b, d0), (t0b + 1, d1)):
            hj = sht_ref[0, 3:4, c, :]
            msg = hj * jnp.exp(-10.0 * d)
            sel = d <= thr
            sacc = sacc + jnp.sum(jnp.where(sel, msg, 0.0),
                                  axis=1, keepdims=True)
            macc = jnp.maximum(
                macc,
                jnp.max(jnp.where(sel, msg, -_BIG), axis=1, keepdims=True))
        return sacc, macc

    def slow_path(_):
        def fill_body(c, carry):
            d2_ref[c] = tile_d2(c)
            return carry

        lax.fori_loop(t0, t1, fill_body, 0)

        prev = jnp.full((_TR, 1), -_BIG, jnp.float32)
        for _ in range(_K):
            def kth_body(c, m, lim=prev):
                tile = d2_ref[c]
                vals = jnp.where(tile > lim, tile, _BIG)
                return jnp.minimum(m, jnp.min(vals, axis=1, keepdims=True))
            prev = lax.fori_loop(t0, t1, kth_body,
                                 jnp.full((_TR, 1), _BIG, jnp.float32))
        thr = prev

        def agg_body(c, carry):
            sacc, macc = carry
            tile = d2_ref[c]
            hj = sht_ref[0, 3:4, c, :]
            msg = hj * jnp.exp(-10.0 * tile)
            sel = tile <= thr
            sacc = sacc + jnp.sum(jnp.where(sel, msg, 0.0),
                                  axis=1, keepdims=True)
            macc = jnp.maximum(
                macc,
                jnp.max(jnp.where(sel, msg, -_BIG), axis=1, keepdims=True))
            return sacc, macc

        return lax.fori_loop(
            t0, t1, agg_body,
            (jnp.zeros((_TR, 1), jnp.float32),
             jnp.full((_TR, 1), -_BIG, jnp.float32)))

    sacc, macc = lax.cond(t1 - jnp.minimum(t0, _NT - 2) <= 2,
                          fast_path, slow_path, 0)

    mean = sacc * (1.0 / _K)
    wo2 = wo2_ref[0]
    v = (jnp.dot(x_ref[0], wo1_ref[0], preferred_element_type=jnp.float32)
         + mean * wo2[0:1, :] + macc * wo2[1:2, :] + bo2_ref[0])
    y_ref[0] = v


def _gravnet_call(shr, sht4, br, bc4, x5, wo1, wo2, bo2):
    return pl.pallas_call(
        _gravnet_body,
        grid=(_NB, _N // _TR),
        in_specs=[
            pl.BlockSpec((1, _TR, 4), lambda b, r: (b, r, 0)),
            pl.BlockSpec((1, 4, _NT, _TCW), lambda b, r: (b, 0, 0, 0)),
            pl.BlockSpec((1, _TR, 1), lambda b, r: (b, r, 0)),
            pl.BlockSpec((1, _NT, 1, _TCW), lambda b, r: (b, 0, 0, 0)),
            pl.BlockSpec((1, _TR, _D), lambda b, r: (b, r, 0)),
            pl.BlockSpec((1, _D, _OUT), lambda b, r: (b, 0, 0)),
            pl.BlockSpec((1, 2, _OUT), lambda b, r: (b, 0, 0)),
            pl.BlockSpec((1, 1, _OUT), lambda b, r: (b, 0, 0)),
        ],
        out_specs=pl.BlockSpec((1, _TR, _OUT), lambda b, r: (b, r, 0)),
        out_shape=jax.ShapeDtypeStruct((_NB, _N, _OUT), jnp.float32),
        scratch_shapes=[pltpu.VMEM((_NT, _TR, _TCW), jnp.float32)],
    )(shr, sht4, br, bc4, x5, wo1, wo2, bo2)


_SC_ROWS = _TOTROWS * _OUT // 128
_RPW_SC = _SC_ROWS // _SC_WORKERS


def _eluw_sc(v2):
    mesh = plsc.VectorSubcoreMesh(core_axis_name="c", subcore_axis_name="s")

    @functools.partial(
        pl.kernel,
        mesh=mesh,
        out_type=jax.ShapeDtypeStruct((_SC_ROWS, 128), jnp.float32),
        scratch_types=[
            pltpu.VMEM((_RPW_SC, 128), jnp.float32),
        ],
    )
    def run(v_hbm, t_hbm, v_v):
        cid = lax.axis_index("c")
        sid = lax.axis_index("s")
        wid = sid * 2 + cid
        base = wid * _RPW_SC
        pltpu.sync_copy(v_hbm.at[pl.ds(base, _RPW_SC)], v_v)

        def body(i, carry):
            for c in range(128 // 16):
                sl = pl.ds(c * 16, 16)
                x = v_v[i, sl]
                v_v[i, sl] = jnp.where(x > 0.0, x, jnp.exp(x) - 1.0)
            return carry

        lax.fori_loop(0, _RPW_SC, body, 0)
        pltpu.sync_copy(v_v, t_hbm.at[pl.ds(base, _RPW_SC)])

    return run(v2)


_TRC = 2048


def _head_body(t_ref, w_ref, oh_ref, b_ref, o_ref):
    r = pl.program_id(0)

    @pl.when(r == 0)
    def _():
        o_ref[...] = jnp.broadcast_to(b_ref[...], (_G, 1))

    z = jnp.dot(t_ref[...], w_ref[0],
                preferred_element_type=jnp.float32)
    contrib = jax.lax.dot_general(
        oh_ref[...], z, (((0,), (0,)), ((), ())),
        preferred_element_type=jnp.float32)
    o_ref[...] = o_ref[...] + contrib


def _head_call(t2, wh, oh, bh):
    return pl.pallas_call(
        _head_body,
        grid=(_TOTROWS // _TRC,),
        in_specs=[
            pl.BlockSpec((_TRC, _OUT), lambda r: (r, 0)),
            pl.BlockSpec((1, _OUT, 1), lambda r: (r // (_N // _TRC), 0, 0)),
            pl.BlockSpec((_TRC, _G), lambda r: (r, 0)),
            pl.BlockSpec((1, 1), lambda r: (0, 0)),
        ],
        out_specs=pl.BlockSpec((_G, 1), lambda r: (0, 0)),
        out_shape=jax.ShapeDtypeStruct((_G, 1), jnp.float32),
    )(t2, wh, oh, bh)


def kernel(x_SiPix, batch_SiPix, x_Crystal, batch_Crystal, x_WSi, batch_WSi,
           x_PbSi, batch_PbSi, x_PbScint, batch_PbScint, params):
    xs = jnp.stack([x_SiPix, x_Crystal, x_WSi, x_PbSi, x_PbScint])
    bs = jnp.stack([batch_SiPix, batch_Crystal, batch_WSi, batch_PbSi,
                    batch_PbScint]).astype(jnp.int32)
    convs = [params['conv1'], params['conv1'], params['conv2'],
             params['conv3'], params['conv4']]
    wsh = jnp.stack([jnp.concatenate([c['Ws'], c['Wh']], axis=1)
                     for c in convs])
    bsh = jnp.stack([jnp.concatenate([c['bs'], c['bh']])[None, :]
                     for c in convs])
    wo1 = jnp.stack([c['Wo1'] for c in convs])
    wo2 = jnp.stack([c['Wo2'] for c in convs])
    bo2 = jnp.stack([c['bo2'][None, :] for c in convs])

    sh = _sh_call(xs, wsh, bsh)
    sht4 = jnp.swapaxes(sh, 1, 2).reshape(_NB, 4, _NT, _TCW)
    br = bs[:, :, None]
    bc4 = bs.reshape(_NB, _NT, 1, _TCW)

    v = _gravnet_call(sh, sht4, br, bc4, xs, wo1, wo2, bo2)

    v2 = v.reshape(_SC_ROWS, 128)
    t2 = _eluw_sc(v2)

    wh = params['out']['W'].reshape(_NB, _OUT, 1)
    oh = (bs.reshape(_TOTROWS)[:, None]
          == jnp.arange(_G, dtype=jnp.int32)[None, :]).astype(jnp.float32)
    bh = params['out']['b'].reshape(1, 1)
    return _head_call(t2.reshape(_TOTROWS, _OUT), wh, oh, bh)


# speedup vs baseline: 1.6058x; 1.0013x over previous
import functools

import jax
import jax.numpy as jnp
from jax import lax
from jax.experimental import pallas as pl
from jax.experimental.pallas import tpu as pltpu
from jax.experimental.pallas import tpu_sc as plsc

_N = 4096
_D = 128
_G = 32
_K = 16
_OUT = 64
_NB = 5
_TR = 256
_TCW = 256
_NT = _N // _TCW
_BIG = 3.0e38

_TOTROWS = _NB * _N
_SC_CHUNK = 128
_SC_WORKERS = 32
_RPW = _TOTROWS // _SC_WORKERS


def _sh_body(x_ref, w_ref, b_ref, o_ref):
    o_ref[0] = (
        jnp.dot(x_ref[0], w_ref[0], preferred_element_type=jnp.float32)
        + b_ref[0]
    )


def _sh_call(x5, wsh, bsh):
    return pl.pallas_call(
        _sh_body,
        grid=(_NB,),
        in_specs=[
            pl.BlockSpec((1, _N, _D), lambda b: (b, 0, 0)),
            pl.BlockSpec((1, _D, 4), lambda b: (b, 0, 0)),
            pl.BlockSpec((1, 1, 4), lambda b: (b, 0, 0)),
        ],
        out_specs=pl.BlockSpec((1, _N, 4), lambda b: (b, 0, 0)),
        out_shape=jax.ShapeDtypeStruct((_NB, _N, 4), jnp.float32),
    )(x5, wsh, bsh)


def _gravnet_body(shr_ref, sht_ref, br_ref, bc_ref, x_ref, wo1_ref, wo2_ref,
                  bo2_ref, y_ref, d2_ref):
    shr = shr_ref[0]
    bi = br_ref[0]
    six = shr[:, 0:1]
    siy = shr[:, 1:2]
    siz = shr[:, 2:3]

    bcall = bc_ref[0]
    lo_g = jnp.min(bi)
    hi_g = jnp.max(bi)
    col_lo = jnp.sum((bcall < lo_g).astype(jnp.int32))
    col_hi = jnp.sum((bcall <= hi_g).astype(jnp.int32))
    t0 = col_lo // _TCW
    t1 = (col_hi + _TCW - 1) // _TCW

    def tile_d2(c):
        sjx = sht_ref[0, 0:1, c, :]
        sjy = sht_ref[0, 1:2, c, :]
        sjz = sht_ref[0, 2:3, c, :]
        bj = bc_ref[0, c]
        dx = six - sjx
        dy = siy - sjy
        dz = siz - sjz
        d2 = (dx * dx + dy * dy) + dz * dz
        return jnp.where(bi != bj, jnp.float32(1e10), d2)

    def fast_path(_):
        t0b = jnp.minimum(t0, _NT - 2)
        d0 = tile_d2(t0b)
        d1 = tile_d2(t0b + 1)
        prev = jnp.full((_TR, 1), -_BIG, jnp.float32)
        for _ in range(_K):
            v0 = jnp.min(jnp.where(d0 > prev, d0, _BIG),
                         axis=1, keepdims=True)
            v1 = jnp.min(jnp.where(d1 > prev, d1, _BIG),
                         axis=1, keepdims=True)
            prev = jnp.minimum(v0, v1)
        thr = prev
        sacc = jnp.zeros((_TR, 1), jnp.float32)
        macc = jnp.full((_TR, 1), -_BIG, jnp.float32)
        for c, d in ((t0b, d0), (t0b + 1, d1)):
            hj = sht_ref[0, 3:4, c, :]
            msg = hj * jnp.exp(-10.0 * d)
            sel = d <= thr
            sacc = sacc + jnp.sum(jnp.where(sel, msg, 0.0),
                                  axis=1, keepdims=True)
            macc = jnp.maximum(
                macc,
                jnp.max(jnp.where(sel, msg, -_BIG), axis=1, keepdims=True))
        return sacc, macc

    def slow_path(_):
        def fill_body(c, carry):
            d2_ref[c] = tile_d2(c)
            return carry

        lax.fori_loop(t0, t1, fill_body, 0)

        prev = jnp.full((_TR, 1), -_BIG, jnp.float32)
        for _ in range(_K):
            def kth_body(c, m, lim=prev):
                tile = d2_ref[c]
                vals = jnp.where(tile > lim, tile, _BIG)
                return jnp.minimum(m, jnp.min(vals, axis=1, keepdims=True))
            prev = lax.fori_loop(t0, t1, kth_body,
                                 jnp.full((_TR, 1), _BIG, jnp.float32))
        thr = prev

        def agg_body(c, carry):
            sacc, macc = carry
            tile = d2_ref[c]
            hj = sht_ref[0, 3:4, c, :]
            msg = hj * jnp.exp(-10.0 * tile)
            sel = tile <= thr
            sacc = sacc + jnp.sum(jnp.where(sel, msg, 0.0),
                                  axis=1, keepdims=True)
            macc = jnp.maximum(
                macc,
                jnp.max(jnp.where(sel, msg, -_BIG), axis=1, keepdims=True))
            return sacc, macc

        return lax.fori_loop(
            t0, t1, agg_body,
            (jnp.zeros((_TR, 1), jnp.float32),
             jnp.full((_TR, 1), -_BIG, jnp.float32)))

    sacc, macc = lax.cond(t1 - jnp.minimum(t0, _NT - 2) <= 2,
                          fast_path, slow_path, 0)

    mean = sacc * (1.0 / _K)
    wo2 = wo2_ref[0]
    v = (jnp.dot(x_ref[0], wo1_ref[0], preferred_element_type=jnp.float32)
         + mean * wo2[0:1, :] + macc * wo2[1:2, :] + bo2_ref[0])
    y_ref[0] = v


def _gravnet_call(shr, sht4, br, bc4, x5, wo1, wo2, bo2):
    return pl.pallas_call(
        _gravnet_body,
        grid=(_NB, _N // _TR),
        in_specs=[
            pl.BlockSpec((1, _TR, 4), lambda b, r: (b, r, 0)),
            pl.BlockSpec((1, 4, _NT, _TCW), lambda b, r: (b, 0, 0, 0)),
            pl.BlockSpec((1, _TR, 1), lambda b, r: (b, r, 0)),
            pl.BlockSpec((1, _NT, 1, _TCW), lambda b, r: (b, 0, 0, 0)),
            pl.BlockSpec((1, _TR, _D), lambda b, r: (b, r, 0)),
            pl.BlockSpec((1, _D, _OUT), lambda b, r: (b, 0, 0)),
            pl.BlockSpec((1, 2, _OUT), lambda b, r: (b, 0, 0)),
            pl.BlockSpec((1, 1, _OUT), lambda b, r: (b, 0, 0)),
        ],
        out_specs=pl.BlockSpec((1, _TR, _OUT), lambda b, r: (b, r, 0)),
        out_shape=jax.ShapeDtypeStruct((_NB, _N, _OUT), jnp.float32),
        scratch_shapes=[pltpu.VMEM((_NT, _TR, _TCW), jnp.float32)],
        compiler_params=pltpu.CompilerParams(
            dimension_semantics=("parallel", "arbitrary")),
    )(shr, sht4, br, bc4, x5, wo1, wo2, bo2)


_SC_ROWS = _TOTROWS * _OUT // 128
_RPW_SC = _SC_ROWS // _SC_WORKERS


def _eluw_sc(v2):
    mesh = plsc.VectorSubcoreMesh(core_axis_name="c", subcore_axis_name="s")

    @functools.partial(
        pl.kernel,
        mesh=mesh,
        out_type=jax.ShapeDtypeStruct((_SC_ROWS, 128), jnp.float32),
        scratch_types=[
            pltpu.VMEM((_RPW_SC, 128), jnp.float32),
        ],
    )
    def run(v_hbm, t_hbm, v_v):
        cid = lax.axis_index("c")
        sid = lax.axis_index("s")
        wid = sid * 2 + cid
        base = wid * _RPW_SC
        pltpu.sync_copy(v_hbm.at[pl.ds(base, _RPW_SC)], v_v)

        def body(i, carry):
            for c in range(128 // 16):
                sl = pl.ds(c * 16, 16)
                x = v_v[i, sl]
                v_v[i, sl] = jnp.where(x > 0.0, x, jnp.exp(x) - 1.0)
            return carry

        lax.fori_loop(0, _RPW_SC, body, 0)
        pltpu.sync_copy(v_v, t_hbm.at[pl.ds(base, _RPW_SC)])

    return run(v2)


_TRC = 2048


def _head_body(t_ref, w_ref, oh_ref, b_ref, o_ref):
    r = pl.program_id(0)

    @pl.when(r == 0)
    def _():
        o_ref[...] = jnp.broadcast_to(b_ref[...], (_G, 1))

    z = jnp.dot(t_ref[...], w_ref[0],
                preferred_element_type=jnp.float32)
    contrib = jax.lax.dot_general(
        oh_ref[...], z, (((0,), (0,)), ((), ())),
        preferred_element_type=jnp.float32)
    o_ref[...] = o_ref[...] + contrib


def _head_call(t2, wh, oh, bh):
    return pl.pallas_call(
        _head_body,
        grid=(_TOTROWS // _TRC,),
        in_specs=[
            pl.BlockSpec((_TRC, _OUT), lambda r: (r, 0)),
            pl.BlockSpec((1, _OUT, 1), lambda r: (r // (_N // _TRC), 0, 0)),
            pl.BlockSpec((_TRC, _G), lambda r: (r, 0)),
            pl.BlockSpec((1, 1), lambda r: (0, 0)),
        ],
        out_specs=pl.BlockSpec((_G, 1), lambda r: (0, 0)),
        out_shape=jax.ShapeDtypeStruct((_G, 1), jnp.float32),
    )(t2, wh, oh, bh)


def kernel(x_SiPix, batch_SiPix, x_Crystal, batch_Crystal, x_WSi, batch_WSi,
           x_PbSi, batch_PbSi, x_PbScint, batch_PbScint, params):
    xs = jnp.stack([x_SiPix, x_Crystal, x_WSi, x_PbSi, x_PbScint])
    bs = jnp.stack([batch_SiPix, batch_Crystal, batch_WSi, batch_PbSi,
                    batch_PbScint]).astype(jnp.int32)
    convs = [params['conv1'], params['conv1'], params['conv2'],
             params['conv3'], params['conv4']]
    wsh = jnp.stack([jnp.concatenate([c['Ws'], c['Wh']], axis=1)
                     for c in convs])
    bsh = jnp.stack([jnp.concatenate([c['bs'], c['bh']])[None, :]
                     for c in convs])
    wo1 = jnp.stack([c['Wo1'] for c in convs])
    wo2 = jnp.stack([c['Wo2'] for c in convs])
    bo2 = jnp.stack([c['bo2'][None, :] for c in convs])

    sh = _sh_call(xs, wsh, bsh)
    sht4 = jnp.swapaxes(sh, 1, 2).reshape(_NB, 4, _NT, _TCW)
    br = bs[:, :, None]
    bc4 = bs.reshape(_NB, _NT, 1, _TCW)

    v = _gravnet_call(sh, sht4, br, bc4, xs, wo1, wo2, bo2)

    v2 = v.reshape(_SC_ROWS, 128)
    t2 = _eluw_sc(v2)

    wh = params['out']['W'].reshape(_NB, _OUT, 1)
    oh = (bs.reshape(_TOTROWS)[:, None]
          == jnp.arange(_G, dtype=jnp.int32)[None, :]).astype(jnp.float32)
    bh = params['out']['b'].reshape(1, 1)
    return _head_call(t2.reshape(_TOTROWS, _OUT), wh, oh, bh)
